# Initial kernel scaffold; baseline (speedup 1.0000x reference)
import jax
import jax.numpy as jnp
from jax.experimental import pallas as pl


def kernel(hidden_states, router_logits, gate_up_proj, gate_up_proj_bias, down_proj, down_proj_bias):
    raise NotImplementedError("write your pallas kernel here")


# speedup vs baseline: 4.4189x; 4.4189x over previous
import jax
import jax.numpy as jnp
from jax.experimental import pallas as pl

ALPHA = 1.702
LIMIT = 7.0


def _moe_body(hs_ref, rl_ref, w1g_ref, w1u_ref, bg_ref, bu_ref, w2_ref,
              b2_ref, out_ref):
    e = pl.program_id(0)
    hs = hs_ref[...]
    gate = jnp.dot(hs, w1g_ref[0], preferred_element_type=jnp.float32)
    up = jnp.dot(hs, w1u_ref[0], preferred_element_type=jnp.float32)
    gate = jnp.minimum(gate + bg_ref[0], LIMIT)
    up = jnp.clip(up + bu_ref[0], -LIMIT, LIMIT)
    glu = gate * jax.nn.sigmoid(gate * ALPHA)
    act = ((up + 1.0) * glu).astype(jnp.bfloat16)
    h = jnp.dot(act, w2_ref[0], preferred_element_type=jnp.float32)
    h = h + b2_ref[0]
    rl = rl_ref[...]
    lane = jax.lax.broadcasted_iota(jnp.int32, rl.shape, 1)
    rw = jnp.sum(jnp.where(lane == e, rl, 0.0), axis=1, keepdims=True)
    contrib = h * rw

    @pl.when(e == 0)
    def _init():
        out_ref[...] = contrib

    @pl.when(e != 0)
    def _acc():
        out_ref[...] = out_ref[...] + contrib


def kernel(hidden_states, router_logits, gate_up_proj, gate_up_proj_bias,
           down_proj, down_proj_bias):
    orig_shape = hidden_states.shape
    H = orig_shape[-1]
    E = router_logits.shape[1]
    I = down_proj.shape[1]
    hs = hidden_states.reshape(-1, H).astype(jnp.bfloat16)
    T = hs.shape[0]
    w1g = gate_up_proj[:, :, 0::2].astype(jnp.bfloat16)
    w1u = gate_up_proj[:, :, 1::2].astype(jnp.bfloat16)
    bg = gate_up_proj_bias[:, 0::2].reshape(E, 1, I)
    bu = gate_up_proj_bias[:, 1::2].reshape(E, 1, I)
    w2 = down_proj.astype(jnp.bfloat16)
    b2 = down_proj_bias.reshape(E, 1, H)

    out = pl.pallas_call(
        _moe_body,
        grid=(E,),
        in_specs=[
            pl.BlockSpec((T, H), lambda e: (0, 0)),
            pl.BlockSpec((T, E), lambda e: (0, 0)),
            pl.BlockSpec((1, H, I), lambda e: (e, 0, 0)),
            pl.BlockSpec((1, H, I), lambda e: (e, 0, 0)),
            pl.BlockSpec((1, 1, I), lambda e: (e, 0, 0)),
            pl.BlockSpec((1, 1, I), lambda e: (e, 0, 0)),
            pl.BlockSpec((1, I, H), lambda e: (e, 0, 0)),
            pl.BlockSpec((1, 1, H), lambda e: (e, 0, 0)),
        ],
        out_specs=pl.BlockSpec((T, H), lambda e: (0, 0)),
        out_shape=jax.ShapeDtypeStruct((T, H), jnp.float32),
    )(hs, router_logits, w1g, w1u, bg, bu, w2, b2)
    return out.reshape(orig_shape)


# speedup vs baseline: 11.3012x; 2.5575x over previous
import jax
import jax.numpy as jnp
from jax.experimental import pallas as pl
from jax.experimental.pallas import tpu as pltpu

ALPHA = 1.702
LIMIT = 7.0


def _moe_body(hs_ref, rl_ref, w1_ref, b1_ref, w2x_ref, b2_ref, out_ref):
    e = pl.program_id(0)
    hs = hs_ref[...]
    gu = jnp.dot(hs, w1_ref[0], preferred_element_type=jnp.float32)
    gu = gu + b1_ref[0]
    gub = gu.astype(jnp.bfloat16)
    up_al = pltpu.roll(gub, gub.shape[1] - 1, 1)
    gate = jnp.minimum(gub, jnp.bfloat16(LIMIT))
    up = jnp.clip(up_al, jnp.bfloat16(-LIMIT), jnp.bfloat16(LIMIT))
    glu = gate * jax.nn.sigmoid(gate * jnp.bfloat16(ALPHA))
    act = (up + jnp.bfloat16(1.0)) * glu
    h = jnp.dot(act, w2x_ref[0], preferred_element_type=jnp.float32)
    h = h + b2_ref[0]
    rl = rl_ref[...]
    lane = jax.lax.broadcasted_iota(jnp.int32, rl.shape, 1)
    rw = jnp.sum(jnp.where(lane == e, rl, 0.0), axis=1, keepdims=True)
    contrib = h * rw

    @pl.when(e == 0)
    def _init():
        out_ref[...] = contrib

    @pl.when(e != 0)
    def _acc():
        out_ref[...] = out_ref[...] + contrib


def kernel(hidden_states, router_logits, gate_up_proj, gate_up_proj_bias,
           down_proj, down_proj_bias):
    orig_shape = hidden_states.shape
    H = orig_shape[-1]
    E = router_logits.shape[1]
    I = down_proj.shape[1]
    F = gate_up_proj.shape[2]
    hs = hidden_states.reshape(-1, H).astype(jnp.bfloat16)
    T = hs.shape[0]
    w1 = gate_up_proj.astype(jnp.bfloat16)
    b1 = gate_up_proj_bias.reshape(E, 1, F)
    w2b = down_proj.astype(jnp.bfloat16)
    w2x = jnp.stack([w2b, jnp.zeros_like(w2b)], axis=2).reshape(E, F, H)
    b2 = down_proj_bias.reshape(E, 1, H)

    out = pl.pallas_call(
        _moe_body,
        grid=(E,),
        in_specs=[
            pl.BlockSpec((T, H), lambda e: (0, 0)),
            pl.BlockSpec((T, E), lambda e: (0, 0)),
            pl.BlockSpec((1, H, F), lambda e: (e, 0, 0)),
            pl.BlockSpec((1, 1, F), lambda e: (e, 0, 0)),
            pl.BlockSpec((1, F, H), lambda e: (e, 0, 0)),
            pl.BlockSpec((1, 1, H), lambda e: (e, 0, 0)),
        ],
        out_specs=pl.BlockSpec((T, H), lambda e: (0, 0)),
        out_shape=jax.ShapeDtypeStruct((T, H), jnp.float32),
    )(hs, router_logits, w1, b1, w2x, b2)
    return out.reshape(orig_shape)


# speedup vs baseline: 20.4468x; 1.8093x over previous
import jax
import jax.numpy as jnp
from jax.experimental import pallas as pl

ALPHA = 1.702
LIMIT = 7.0


def _moe_body(hs_ref, rl_ref, w1g_ref, w1u_ref, bg_ref, bu_ref, w2_ref,
              b2_ref, out_ref):
    e = pl.program_id(0)
    hs = hs_ref[...]
    gate = jnp.dot(hs, w1g_ref[0], preferred_element_type=jnp.float32)
    up = jnp.dot(hs, w1u_ref[0], preferred_element_type=jnp.float32)
    gate = (gate + bg_ref[0]).astype(jnp.bfloat16)
    up = (up + bu_ref[0]).astype(jnp.bfloat16)
    gate = jnp.minimum(gate, jnp.bfloat16(LIMIT))
    up = jnp.clip(up, jnp.bfloat16(-LIMIT), jnp.bfloat16(LIMIT))
    glu = gate * jax.nn.sigmoid(gate * jnp.bfloat16(ALPHA))
    act = (up + jnp.bfloat16(1.0)) * glu
    h = jnp.dot(act, w2_ref[0], preferred_element_type=jnp.float32)
    h = h + b2_ref[0]
    rl = rl_ref[...]
    lane = jax.lax.broadcasted_iota(jnp.int32, rl.shape, 1)
    rw = jnp.sum(jnp.where(lane == e, rl, 0.0), axis=1, keepdims=True)
    contrib = h * rw

    @pl.when(e == 0)
    def _init():
        out_ref[...] = contrib

    @pl.when(e != 0)
    def _acc():
        out_ref[...] = out_ref[...] + contrib


def kernel(hidden_states, router_logits, gate_up_proj, gate_up_proj_bias,
           down_proj, down_proj_bias):
    orig_shape = hidden_states.shape
    H = orig_shape[-1]
    E = router_logits.shape[1]
    I = down_proj.shape[1]
    hs = hidden_states.reshape(-1, H).astype(jnp.bfloat16)
    T = hs.shape[0]
    w1r = gate_up_proj.reshape(E, H, I, 2).astype(jnp.bfloat16)
    w1g = w1r[..., 0]
    w1u = w1r[..., 1]
    b1r = gate_up_proj_bias.reshape(E, 1, I, 2)
    bg = b1r[..., 0]
    bu = b1r[..., 1]
    w2 = down_proj.astype(jnp.bfloat16)
    b2 = down_proj_bias.reshape(E, 1, H)

    out = pl.pallas_call(
        _moe_body,
        grid=(E,),
        in_specs=[
            pl.BlockSpec((T, H), lambda e: (0, 0)),
            pl.BlockSpec((T, E), lambda e: (0, 0)),
            pl.BlockSpec((1, H, I), lambda e: (e, 0, 0)),
            pl.BlockSpec((1, H, I), lambda e: (e, 0, 0)),
            pl.BlockSpec((1, 1, I), lambda e: (e, 0, 0)),
            pl.BlockSpec((1, 1, I), lambda e: (e, 0, 0)),
            pl.BlockSpec((1, I, H), lambda e: (e, 0, 0)),
            pl.BlockSpec((1, 1, H), lambda e: (e, 0, 0)),
        ],
        out_specs=pl.BlockSpec((T, H), lambda e: (0, 0)),
        out_shape=jax.ShapeDtypeStruct((T, H), jnp.float32),
    )(hs, router_logits, w1g, w1u, bg, bu, w2, b2)
    return out.reshape(orig_shape)


# speedup vs baseline: 21.3262x; 1.0430x over previous
import jax
import jax.numpy as jnp
from jax.experimental import pallas as pl

ALPHA = 1.702
LIMIT = 7.0


def _glu(gate_f32, up_f32, rw):
    gate = gate_f32.astype(jnp.bfloat16)
    up = up_f32.astype(jnp.bfloat16)
    gate = jnp.minimum(gate, jnp.bfloat16(LIMIT))
    up = jnp.clip(up, jnp.bfloat16(-LIMIT), jnp.bfloat16(LIMIT))
    glu = gate * jax.nn.sigmoid(gate * jnp.bfloat16(ALPHA))
    return (up + jnp.bfloat16(1.0)) * glu * rw


def _moe_body(hs_ref, rl_ref, b2_ref, w1g_ref, w1u_ref, bg_ref, bu_ref,
              w2p_ref, out_ref):
    p = pl.program_id(0)
    hs = hs_ref[...]
    rl = rl_ref[...]
    lane = jax.lax.broadcasted_iota(jnp.int32, rl.shape, 1)
    rw_a = jnp.sum(jnp.where(lane == 2 * p, rl, 0.0), axis=1, keepdims=True)
    rw_b = jnp.sum(jnp.where(lane == 2 * p + 1, rl, 0.0), axis=1,
                   keepdims=True)

    gate_a = jnp.dot(hs, w1g_ref[0], preferred_element_type=jnp.float32)
    up_a = jnp.dot(hs, w1u_ref[0], preferred_element_type=jnp.float32)
    act_a = _glu(gate_a + bg_ref[0, 0], up_a + bu_ref[0, 0],
                 rw_a.astype(jnp.bfloat16))
    gate_b = jnp.dot(hs, w1g_ref[1], preferred_element_type=jnp.float32)
    up_b = jnp.dot(hs, w1u_ref[1], preferred_element_type=jnp.float32)
    act_b = _glu(gate_b + bg_ref[1, 0], up_b + bu_ref[1, 0],
                 rw_b.astype(jnp.bfloat16))

    cat = jnp.concatenate([act_a, act_b], axis=1)
    h = jnp.dot(cat, w2p_ref[0], preferred_element_type=jnp.float32)

    @pl.when(p == 0)
    def _init():
        out_ref[...] = h + jnp.dot(rl, b2_ref[...],
                                   preferred_element_type=jnp.float32)

    @pl.when(p != 0)
    def _acc():
        out_ref[...] = out_ref[...] + h


def kernel(hidden_states, router_logits, gate_up_proj, gate_up_proj_bias,
           down_proj, down_proj_bias):
    orig_shape = hidden_states.shape
    H = orig_shape[-1]
    E = router_logits.shape[1]
    I = down_proj.shape[1]
    hs = hidden_states.reshape(-1, H).astype(jnp.bfloat16)
    T = hs.shape[0]
    w1r = gate_up_proj.reshape(E, H, I, 2).astype(jnp.bfloat16)
    w1g = w1r[..., 0]
    w1u = w1r[..., 1]
    b1r = gate_up_proj_bias.reshape(E, 1, I, 2)
    bg = b1r[..., 0]
    bu = b1r[..., 1]
    w2p = down_proj.reshape(E // 2, 2 * I, H).astype(jnp.bfloat16)

    out = pl.pallas_call(
        _moe_body,
        grid=(E // 2,),
        in_specs=[
            pl.BlockSpec((T, H), lambda p: (0, 0)),
            pl.BlockSpec((T, E), lambda p: (0, 0)),
            pl.BlockSpec((E, H), lambda p: (0, 0)),
            pl.BlockSpec((2, H, I), lambda p: (p, 0, 0)),
            pl.BlockSpec((2, H, I), lambda p: (p, 0, 0)),
            pl.BlockSpec((2, 1, I), lambda p: (p, 0, 0)),
            pl.BlockSpec((2, 1, I), lambda p: (p, 0, 0)),
            pl.BlockSpec((1, 2 * I, H), lambda p: (p, 0, 0)),
        ],
        out_specs=pl.BlockSpec((T, H), lambda p: (0, 0)),
        out_shape=jax.ShapeDtypeStruct((T, H), jnp.float32),
    )(hs, router_logits, down_proj_bias, w1g, w1u, bg, bu, w2p)
    return out.reshape(orig_shape)
